# baseline (speedup 1.0000x reference)
import functools

import jax
import jax.numpy as jnp
from jax import lax
from jax.experimental import pallas as pl
from jax.experimental.pallas import tpu as pltpu
from jax.experimental.pallas import tpu_sc as plsc

VOCAB = 1000000
D = 64
R = 16384
C = 50
NC, NS = 2, 16
NW = NC * NS
CH = 128
N_CHUNKS = R * C // CH
CH_PER_W = N_CHUNKS // NW
RB = R // CH
NBUF = 4
SCALE = 8.0

_mesh = plsc.VectorSubcoreMesh(
    core_axis_name="c", subcore_axis_name="s", num_cores=NC, num_subcores=NS
)


@functools.partial(
    pl.kernel,
    out_type=jax.ShapeDtypeStruct((C, D // 8, RB, 8 * CH), jnp.float32),
    mesh=_mesh,
    scratch_types=[
        pltpu.VMEM((CH_PER_W, CH), jnp.int32),
        pltpu.VMEM((NBUF * CH, D), jnp.float32),
        pltpu.VMEM((2 * D * CH,), jnp.float32),
        pltpu.SemaphoreType.DMA,
        pltpu.SemaphoreType.DMA,
    ],
    compiler_params=pltpu.CompilerParams(
        use_tc_tiling_on_sc=False, needs_layout_passes=False
    ),
)
def _emb_kernel(idx_hbm, table_hbm, out_hbm, idx_v, bufs, obuf, gsem, ssem):
    wid = lax.axis_index("s") * NC + lax.axis_index("c")
    k0 = wid * CH_PER_W

    pltpu.sync_copy(idx_hbm.at[pl.ds(k0, CH_PER_W)], idx_v)

    for b in range(NBUF - 1):
        pltpu.async_copy(
            table_hbm.at[idx_v.at[b]], bufs.at[pl.ds(b * CH, CH)], gsem
        )

    iota = lax.iota(jnp.int32, 16)
    rot = [lax.bitwise_and(iota + k, 15) for k in range(16)]
    srot = [r * CH + iota for r in rot]

    def do_chunk(j, b):
        jj = j % 2
        k = k0 + j
        col = k // RB
        rb = k % RB

        @pl.when(j + NBUF - 1 < CH_PER_W)
        def _():
            pltpu.async_copy(
                table_hbm.at[idx_v.at[j + NBUF - 1]],
                bufs.at[pl.ds(((b + NBUF - 1) % NBUF) * CH, CH)],
                gsem,
            )

        @pl.when(j >= 2)
        def _():
            for db in range(D // 8):
                pltpu.make_async_copy(
                    obuf.at[pl.ds(0, 8 * CH)], out_hbm.at[0, db, 0], ssem
                ).wait()

        pltpu.make_async_copy(
            table_hbm.at[pl.ds(0, CH)], bufs.at[pl.ds(0, CH)], gsem
        ).wait()

        for d0 in range(0, D, 16):
            dvecs = [rv + d0 for rv in rot]
            srots = [sv + (jj * D * CH + d0 * CH) for sv in srot]

            def tr_rblock(rblk, carry, dvecs=dvecs, srots=srots):
                r0 = rblk * 16
                rvec = iota + (b * CH + r0)
                for kk in range(16):
                    vals = plsc.load_gather(bufs, [rvec, dvecs[kk]])
                    obuf_idx = srots[kk] + r0
                    plsc.store_scatter(obuf, [obuf_idx], vals * SCALE)
                return carry

            lax.fori_loop(0, CH // 16, tr_rblock, 0)

        for db in range(D // 8):
            pltpu.async_copy(
                obuf.at[pl.ds(jj * D * CH + db * 8 * CH, 8 * CH)],
                out_hbm.at[col, db, rb],
                ssem,
            )

    def outer(g, carry):
        for b in range(NBUF):
            do_chunk(g * NBUF + b, b)
        return carry

    lax.fori_loop(0, CH_PER_W // NBUF, outer, 0)

    for _ in range(2 * (D // 8)):
        pltpu.make_async_copy(
            obuf.at[pl.ds(0, 8 * CH)], out_hbm.at[0, 0, 0], ssem
        ).wait()


def kernel(x, table):
    idx = x.T.astype(jnp.int32).reshape(N_CHUNKS, CH)
    out4 = _emb_kernel(idx, table)
    out5 = out4.reshape(C, D // 8, RB, 8, CH)
    return out5.transpose(2, 4, 0, 1, 3).reshape(R, C, D)


# speedup vs baseline: 1.1892x; 1.1892x over previous
import functools

import jax
import jax.numpy as jnp
from jax import lax
from jax.experimental import pallas as pl
from jax.experimental.pallas import tpu as pltpu
from jax.experimental.pallas import tpu_sc as plsc

VOCAB = 1000000
D = 64
R = 16384
C = 50
NC, NS = 2, 16
NW = NC * NS
CH = 128
N_CHUNKS = R * C // CH
CH_PER_W = N_CHUNKS // NW
RB = R // CH
NBUF = 4
SCALE = 8.0

_mesh = plsc.VectorSubcoreMesh(
    core_axis_name="c", subcore_axis_name="s", num_cores=NC, num_subcores=NS
)


@functools.partial(
    pl.kernel,
    out_type=jax.ShapeDtypeStruct((C, D // 8, RB, 8 * CH), jnp.float32),
    mesh=_mesh,
    scratch_types=[
        pltpu.VMEM((CH_PER_W, CH), jnp.int32),
        pltpu.VMEM((NBUF * CH, D), jnp.float32),
        pltpu.VMEM((2 * D * CH,), jnp.float32),
        pltpu.SemaphoreType.DMA,
        pltpu.SemaphoreType.DMA,
    ],
    compiler_params=pltpu.CompilerParams(
        use_tc_tiling_on_sc=False, needs_layout_passes=False
    ),
)
def _emb_kernel(idx_hbm, table_hbm, out_hbm, idx_v, bufs, obuf, gsem, ssem):
    wid = lax.axis_index("s") * NC + lax.axis_index("c")
    k0 = wid * CH_PER_W

    pltpu.sync_copy(idx_hbm.at[pl.ds(k0, CH_PER_W)], idx_v)

    for b in range(NBUF - 1):
        pltpu.async_copy(
            table_hbm.at[idx_v.at[b]], bufs.at[pl.ds(b * CH, CH)], gsem
        )

    iota = lax.iota(jnp.int32, 16)
    rot = [lax.bitwise_and(iota + k, 15) for k in range(16)]
    srot = [r * CH + iota for r in rot]

    def do_chunk(j, b):
        jj = j % 2
        k = k0 + j
        col = k // RB
        rb = k % RB

        @pl.when(j + NBUF - 1 < CH_PER_W)
        def _():
            pltpu.async_copy(
                table_hbm.at[idx_v.at[j + NBUF - 1]],
                bufs.at[pl.ds(((b + NBUF - 1) % NBUF) * CH, CH)],
                gsem,
            )

        @pl.when(j >= 2)
        def _():
            for db in range(D // 8):
                pltpu.make_async_copy(
                    obuf.at[pl.ds(0, 8 * CH)], out_hbm.at[0, db, 0], ssem
                ).wait()

        pltpu.make_async_copy(
            table_hbm.at[pl.ds(0, CH)], bufs.at[pl.ds(0, CH)], gsem
        ).wait()

        for d0 in range(0, D, 16):
            dvecs = [rv + d0 for rv in rot]
            srots = [sv + (jj * D * CH + d0 * CH) for sv in srot]

            def tr_rblock(rblk, dvecs=dvecs, srots=srots):
                r0 = rblk * 16
                rvec = iota + (b * CH + r0)
                for kk in range(16):
                    vals = plsc.load_gather(bufs, [rvec, dvecs[kk]])
                    obuf_idx = srots[kk] + r0
                    plsc.store_scatter(obuf, [obuf_idx], vals * SCALE)

            plsc.parallel_loop(0, CH // 16)(tr_rblock)

        for db in range(D // 8):
            pltpu.async_copy(
                obuf.at[pl.ds(jj * D * CH + db * 8 * CH, 8 * CH)],
                out_hbm.at[col, db, rb],
                ssem,
            )

    def outer(g, carry):
        for b in range(NBUF):
            do_chunk(g * NBUF + b, b)
        return carry

    lax.fori_loop(0, CH_PER_W // NBUF, outer, 0)

    for _ in range(2 * (D // 8)):
        pltpu.make_async_copy(
            obuf.at[pl.ds(0, 8 * CH)], out_hbm.at[0, 0, 0], ssem
        ).wait()


def kernel(x, table):
    idx = x.T.astype(jnp.int32).reshape(N_CHUNKS, CH)
    out4 = _emb_kernel(idx, table)
    out5 = out4.reshape(C, D // 8, RB, 8, CH)
    return out5.transpose(2, 4, 0, 1, 3).reshape(R, C, D)


# speedup vs baseline: 1.4839x; 1.2478x over previous
import functools

import jax
import jax.numpy as jnp
from jax import lax
from jax.experimental import pallas as pl
from jax.experimental.pallas import tpu as pltpu
from jax.experimental.pallas import tpu_sc as plsc

VOCAB = 1000000
D = 64
R = 16384
C = 50
NC, NS = 2, 16
NW = NC * NS
CH = 128
N_CHUNKS = R * C // CH
CH_PER_W = N_CHUNKS // NW
RB = R // CH
NBUF = 4
SCALE = 8.0

_mesh = plsc.VectorSubcoreMesh(
    core_axis_name="c", subcore_axis_name="s", num_cores=NC, num_subcores=NS
)


@functools.partial(
    pl.kernel,
    out_type=jax.ShapeDtypeStruct((C, D // 8, RB, 8 * CH), jnp.float32),
    mesh=_mesh,
    scratch_types=[
        pltpu.VMEM((CH_PER_W, CH), jnp.int32),
        pltpu.VMEM((NBUF * CH, D), jnp.float32),
        pltpu.VMEM((2 * D * CH,), jnp.float32),
        pltpu.SemaphoreType.DMA,
        pltpu.SemaphoreType.DMA,
    ],
    compiler_params=pltpu.CompilerParams(
        use_tc_tiling_on_sc=False, needs_layout_passes=False
    ),
)
def _emb_kernel(idx_hbm, table_hbm, out_hbm, idx_v, bufs, obuf, gsem, ssem):
    wid = lax.axis_index("s") * NC + lax.axis_index("c")
    k0 = wid * CH_PER_W

    pltpu.sync_copy(idx_hbm.at[pl.ds(k0, CH_PER_W)], idx_v)

    for b in range(NBUF - 1):
        pltpu.async_copy(
            table_hbm.at[idx_v.at[b]], bufs.at[pl.ds(b * CH, CH)], gsem
        )

    iota = lax.iota(jnp.int32, 16)
    rot = [lax.bitwise_and(iota + k, 15) for k in range(16)]
    srot = [r * CH + iota for r in rot]

    def do_chunk(j, b):
        jj = j % 2
        k = k0 + j
        col = k // RB
        rb = k % RB

        @pl.when(j + NBUF - 1 < CH_PER_W)
        def _():
            pltpu.async_copy(
                table_hbm.at[idx_v.at[j + NBUF - 1]],
                bufs.at[pl.ds(((b + NBUF - 1) % NBUF) * CH, CH)],
                gsem,
            )

        @pl.when(j >= 2)
        def _():
            for db in range(D // 8):
                pltpu.make_async_copy(
                    obuf.at[pl.ds(0, 8 * CH)], out_hbm.at[0, db, 0], ssem
                ).wait()

        pltpu.make_async_copy(
            table_hbm.at[pl.ds(0, CH)], bufs.at[pl.ds(0, CH)], gsem
        ).wait()

        sbase0 = jj * (D * CH)

        def tr_block(blk):
            r0 = lax.mul(lax.rem(blk, CH // 16), 16)
            d0 = lax.mul(lax.div(blk, CH // 16), 16)
            rvec = iota + (b * CH + r0)
            sbase = sbase0 + d0 * CH + r0
            for kk in range(16):
                vals = plsc.load_gather(bufs, [rvec, rot[kk] + d0])
                plsc.store_scatter(obuf, [srot[kk] + sbase], vals * SCALE)

        plsc.parallel_loop(0, (CH // 16) * (D // 16), unroll=2)(tr_block)

        for db in range(D // 8):
            pltpu.async_copy(
                obuf.at[pl.ds(jj * D * CH + db * 8 * CH, 8 * CH)],
                out_hbm.at[col, db, rb],
                ssem,
            )

    def outer(g, carry):
        for b in range(NBUF):
            do_chunk(g * NBUF + b, b)
        return carry

    lax.fori_loop(0, CH_PER_W // NBUF, outer, 0)

    for _ in range(2 * (D // 8)):
        pltpu.make_async_copy(
            obuf.at[pl.ds(0, 8 * CH)], out_hbm.at[0, 0, 0], ssem
        ).wait()


def kernel(x, table):
    idx = x.T.astype(jnp.int32).reshape(N_CHUNKS, CH)
    out4 = _emb_kernel(idx, table)
    out5 = out4.reshape(C, D // 8, RB, 8, CH)
    return out5.transpose(2, 4, 0, 1, 3).reshape(R, C, D)


# speedup vs baseline: 2.8758x; 1.9380x over previous
import functools

import jax
import jax.numpy as jnp
from jax import lax
from jax.experimental import pallas as pl
from jax.experimental.pallas import tpu as pltpu
from jax.experimental.pallas import tpu_sc as plsc

VOCAB = 1000000
D = 64
R = 16384
C = 50
NC, NS = 2, 16
NW = NC * NS
CH = 128
N_CHUNKS = R * C // CH
CH_PER_W = N_CHUNKS // NW
RB = R // CH
NBUF = 4
SCALE = 8.0

_mesh = plsc.VectorSubcoreMesh(
    core_axis_name="c", subcore_axis_name="s", num_cores=NC, num_subcores=NS
)

TB = 7812
TPW = 245


@functools.partial(
    pl.kernel,
    out_type=jax.ShapeDtypeStruct((500032, 128), jnp.float32),
    mesh=_mesh,
    scratch_types=[
        pltpu.VMEM((NBUF, D, CH), jnp.float32),
        pltpu.VMEM((2 * D, CH), jnp.float32),
        pltpu.SemaphoreType.DMA,
        pltpu.SemaphoreType.DMA,
    ],
    compiler_params=pltpu.CompilerParams(
        use_tc_tiling_on_sc=True, needs_layout_passes=False
    ),
)
def _fmt_kernel(tt_hbm, tail_hbm, out_hbm, ibuf, obuf, isem, osem):
    wid = lax.axis_index("s") * NC + lax.axis_index("c")

    iota = lax.iota(jnp.int32, 16)
    rot = [lax.bitwise_and(iota + k, 15) for k in range(16)]
    par = lax.bitwise_and(iota, 1)
    half = lax.shift_right_logical(iota, 1)
    cbase = [r + par * D for r in rot]

    def fire(t, slot):
        ib = wid + 32 * t

        @pl.when(ib < TB)
        def _():
            src = tt_hbm.at[:, pl.ds(pl.multiple_of(ib * CH, CH), CH)]
            pltpu.async_copy(src, ibuf.at[slot], isem)

    for bslot in range(NBUF - 1):
        fire(bslot, bslot)

    def do_col(t, slot):
        ib = wid + 32 * t
        jj = t % 2

        fire(t + NBUF - 1, (slot + NBUF - 1) % NBUF)

        @pl.when(ib < TB)
        def _():
            @pl.when(t >= 2)
            def _():
                pltpu.make_async_copy(
                    obuf.at[pl.ds(0, D)], out_hbm.at[pl.ds(0, D)], osem
                ).wait()

            pltpu.make_async_copy(
                tt_hbm.at[:, pl.ds(0, CH)], ibuf.at[slot], isem
            ).wait()

            def tr_block(blk):
                l0 = lax.mul(lax.rem(blk, 8), 16)
                d0 = lax.mul(lax.div(blk, 8), 16)
                lvec = iota + l0
                rowvec = half + (jj * D + lax.div(l0, 2))
                for kk in range(16):
                    vals = plsc.load_gather(ibuf.at[slot], [rot[kk] + d0, lvec])
                    plsc.store_scatter(obuf, [rowvec, cbase[kk] + d0], vals)

            plsc.parallel_loop(0, 32, unroll=2)(tr_block)

            pltpu.async_copy(
                obuf.at[pl.ds(jj * D, D)],
                out_hbm.at[pl.ds(pl.multiple_of(ib * D, D), D)],
                osem,
            )

    def outer(g, carry):
        for bslot in range(NBUF):
            do_col(g * NBUF + bslot, bslot)
        return carry

    lax.fori_loop(0, 248 // NBUF, outer, 0)

    for _ in range(2):
        pltpu.make_async_copy(
            obuf.at[pl.ds(0, D)], out_hbm.at[pl.ds(0, D)], osem
        ).wait()

    @pl.when(wid == 0)
    def _():
        pltpu.sync_copy(tail_hbm, ibuf.at[0])

        def tr_tail(blk):
            l0 = lax.mul(lax.rem(blk, 4), 16)
            d0 = lax.mul(lax.div(blk, 4), 16)
            lvec = iota + l0
            rowvec = half + lax.div(l0, 2)
            for kk in range(16):
                vals = plsc.load_gather(ibuf.at[0], [rot[kk] + d0, lvec])
                plsc.store_scatter(obuf, [rowvec, cbase[kk] + d0], vals)

        plsc.parallel_loop(0, 16, unroll=2)(tr_tail)
        pltpu.sync_copy(
            obuf.at[pl.ds(0, D // 2)], out_hbm.at[pl.ds(TB * D, D // 2)]
        )


@functools.partial(
    pl.kernel,
    out_type=jax.ShapeDtypeStruct((C, D // 8, RB, 8 * CH), jnp.float32),
    mesh=_mesh,
    scratch_types=[
        pltpu.VMEM((CH_PER_W, CH), jnp.int32),
        pltpu.VMEM((NBUF * CH, D), jnp.float32),
        pltpu.VMEM((2 * D * CH,), jnp.float32),
        pltpu.SemaphoreType.DMA,
        pltpu.SemaphoreType.DMA,
    ],
    compiler_params=pltpu.CompilerParams(
        use_tc_tiling_on_sc=False, needs_layout_passes=False
    ),
)
def _emb_kernel(idx_hbm, table_hbm, out_hbm, idx_v, bufs, obuf, gsem, ssem):
    wid = lax.axis_index("s") * NC + lax.axis_index("c")
    k0 = wid * CH_PER_W

    pltpu.sync_copy(idx_hbm.at[pl.ds(k0, CH_PER_W)], idx_v)

    for b in range(NBUF - 1):
        pltpu.async_copy(
            table_hbm.at[idx_v.at[b]], bufs.at[pl.ds(b * CH, CH)], gsem
        )

    iota = lax.iota(jnp.int32, 16)
    rot = [lax.bitwise_and(iota + k, 15) for k in range(16)]
    srot = [r * CH + iota for r in rot]

    def do_chunk(j, b):
        jj = j % 2
        k = k0 + j
        col = k // RB
        rb = k % RB

        @pl.when(j + NBUF - 1 < CH_PER_W)
        def _():
            pltpu.async_copy(
                table_hbm.at[idx_v.at[j + NBUF - 1]],
                bufs.at[pl.ds(((b + NBUF - 1) % NBUF) * CH, CH)],
                gsem,
            )

        @pl.when(j >= 2)
        def _():
            for db in range(D // 8):
                pltpu.make_async_copy(
                    obuf.at[pl.ds(0, 8 * CH)], out_hbm.at[0, db, 0], ssem
                ).wait()

        pltpu.make_async_copy(
            table_hbm.at[pl.ds(0, CH)], bufs.at[pl.ds(0, CH)], gsem
        ).wait()

        sbase0 = jj * (D * CH)

        def tr_block(blk):
            r0 = lax.mul(lax.rem(blk, CH // 16), 16)
            d0 = lax.mul(lax.div(blk, CH // 16), 16)
            rvec = iota + (b * CH + r0)
            sbase = sbase0 + d0 * CH + r0
            for kk in range(16):
                vals = plsc.load_gather(bufs, [rvec, rot[kk] + d0])
                plsc.store_scatter(obuf, [srot[kk] + sbase], vals * SCALE)

        plsc.parallel_loop(0, (CH // 16) * (D // 16), unroll=2)(tr_block)

        for db in range(D // 8):
            pltpu.async_copy(
                obuf.at[pl.ds(jj * D * CH + db * 8 * CH, 8 * CH)],
                out_hbm.at[col, db, rb],
                ssem,
            )

    def outer(g, carry):
        for b in range(NBUF):
            do_chunk(g * NBUF + b, b)
        return carry

    lax.fori_loop(0, CH_PER_W // NBUF, outer, 0)

    for _ in range(2 * (D // 8)):
        pltpu.make_async_copy(
            obuf.at[pl.ds(0, 8 * CH)], out_hbm.at[0, 0, 0], ssem
        ).wait()


def kernel(x, table):
    idx = x.T.astype(jnp.int32).reshape(N_CHUNKS, CH)
    tailp = jnp.pad(table[TB * CH:].T, ((0, 0), (0, CH - D)))
    tbl = _fmt_kernel(table.T, tailp).reshape(2 * 500032, D)
    out4 = _emb_kernel(idx, tbl)
    out5 = out4.reshape(C, D // 8, RB, 8, CH)
    return out5.transpose(2, 4, 0, 1, 3).reshape(R, C, D)
